# baseline (speedup 1.0000x reference)
import functools

import jax
import jax.numpy as jnp
from jax import lax
from jax.experimental import pallas as pl
from jax.experimental.pallas import tpu as pltpu
from jax.experimental.pallas import tpu_sc as plsc

B = 16384
EMB = 64
NC = 2
NS = 16
NW = NC * NS
SC_N = 8192
TC_N = B - SC_N
BPW = SC_N // NW
NBODY = BPW // 4
IDXPAD = BPW + 32
PPS = 8
TSTEPS = TC_N // PPS


def _mf_body(u_id, i_id, uembT, ub1, iembT, ib1, mean, out,
             idx_u, idx_i, UW_A, IW_A, UW_B, IW_B,
             Ub_A, Ib_A, Ub_B, Ib_B, P, outv, meanv, semA, semB):
    wid = lax.axis_index("s") * NC + lax.axis_index("c")
    base = wid * BPW

    pltpu.sync_copy(u_id.at[pl.ds(base, BPW)], idx_u.at[pl.ds(0, BPW)])
    pltpu.sync_copy(i_id.at[pl.ds(base, BPW)], idx_i.at[pl.ds(0, BPW)])
    pltpu.sync_copy(mean, meanv.at[pl.ds(0, 1)])

    m16 = meanv[...][0] * 0.0625
    iot = lax.iota(jnp.int32, 16)

    def enq(uv, iv, UWx, IWx, Ubx, Ibx, semx):
        for j in range(2):
            uid = uv[j]
            iid = iv[j]
            cu = pl.multiple_of(lax.shift_right_logical(uid, 7) * 128, 128)
            ci = pl.multiple_of(lax.shift_right_logical(iid, 7) * 128, 128)
            pltpu.async_copy(uembT.at[:, pl.ds(cu, 128)], UWx.at[j], semx)
            pltpu.async_copy(iembT.at[:, pl.ds(ci, 128)], IWx.at[j], semx)
            pltpu.async_copy(ub1.at[pl.ds(cu, 128)],
                             Ubx.at[pl.ds(j * 128, 128)], semx)
            pltpu.async_copy(ib1.at[pl.ds(ci, 128)],
                             Ibx.at[pl.ds(j * 128, 128)], semx)

    def drain(UWx, IWx, Ubx, Ibx, semx):
        for j in range(2):
            pltpu.make_async_copy(
                uembT.at[:, pl.ds(0, 128)], UWx.at[j], semx).wait()
            pltpu.make_async_copy(
                iembT.at[:, pl.ds(0, 128)], IWx.at[j], semx).wait()
            pltpu.make_async_copy(
                ub1.at[pl.ds(0, 128)], Ubx.at[pl.ds(j * 128, 128)], semx).wait()
            pltpu.make_async_copy(
                ib1.at[pl.ds(0, 128)], Ibx.at[pl.ds(j * 128, 128)], semx).wait()

    def comp(p0, uv, iv, UWx, IWx, Ubx, Ibx):
        for j in range(2):
            uid = uv[j]
            iid = iv[j]
            luv = jnp.full((16,), 0, jnp.int32) + lax.bitwise_and(uid, jnp.int32(127))
            liv = jnp.full((16,), 0, jnp.int32) + lax.bitwise_and(iid, jnp.int32(127))
            jv = jnp.full((16,), j, jnp.int32)
            acc = jnp.zeros((16,), jnp.float32)
            for g in range(4):
                fv = g * 16 + iot
                uu = plsc.load_gather(UWx, [jv, fv, luv])
                vv = plsc.load_gather(IWx, [jv, fv, liv])
                acc = acc + uu * vv
            ubv = plsc.load_gather(Ubx, [jnp.full((16,), j * 128, jnp.int32) + luv])
            ibv = plsc.load_gather(Ibx, [jnp.full((16,), j * 128, jnp.int32) + liv])
            acc = acc + (ubv + ibv) * 0.0625 + m16
            P[pl.ds((p0 + j) * 16, 16)] = acc

    enq(idx_u[pl.ds(0, 16)], idx_i[pl.ds(0, 16)],
        UW_A, IW_A, Ub_A, Ib_A, semA)
    enq(idx_u[pl.ds(2, 16)], idx_i[pl.ds(2, 16)],
        UW_B, IW_B, Ub_B, Ib_B, semB)

    def body(k, carry):
        pA = k * 4
        uvA = idx_u[pl.ds(pA, 16)]
        ivA = idx_i[pl.ds(pA, 16)]
        drain(UW_A, IW_A, Ub_A, Ib_A, semA)
        comp(pA, uvA, ivA, UW_A, IW_A, Ub_A, Ib_A)

        @pl.when(k < NBODY - 1)
        def _():
            enq(idx_u[pl.ds(pA + 4, 16)], idx_i[pl.ds(pA + 4, 16)],
                UW_A, IW_A, Ub_A, Ib_A, semA)

        pB = k * 4 + 2
        uvB = idx_u[pl.ds(pB, 16)]
        ivB = idx_i[pl.ds(pB, 16)]
        drain(UW_B, IW_B, Ub_B, Ib_B, semB)
        comp(pB, uvB, ivB, UW_B, IW_B, Ub_B, Ib_B)

        @pl.when(k < NBODY - 1)
        def _():
            enq(idx_u[pl.ds(pB + 4, 16)], idx_i[pl.ds(pB + 4, 16)],
                UW_B, IW_B, Ub_B, Ib_B, semB)

        return carry

    lax.fori_loop(0, NBODY, body, 0)

    def phase2(b, carry):
        pvec = (b * 16 + iot) * 16
        acc = jnp.zeros((16,), jnp.float32)
        for l in range(16):
            acc = acc + plsc.load_gather(P, [pvec + jnp.full((16,), l, jnp.int32)])
        outv[pl.ds(b * 16, 16)] = acc
        return carry

    lax.fori_loop(0, BPW // 16, phase2, 0)

    pltpu.sync_copy(outv, out.at[pl.ds(base, BPW)])


def _sc_call(u_id, i_id, uembT, ub1, iembT, ib1, mean):
    mesh = plsc.VectorSubcoreMesh(core_axis_name="c", subcore_axis_name="s")
    f = functools.partial(
        pl.kernel,
        out_type=jax.ShapeDtypeStruct((SC_N,), jnp.float32),
        mesh=mesh,
        scratch_types=[
            pltpu.VMEM((IDXPAD,), jnp.int32),
            pltpu.VMEM((IDXPAD,), jnp.int32),
            pltpu.VMEM((2, EMB, 128), jnp.float32),
            pltpu.VMEM((2, EMB, 128), jnp.float32),
            pltpu.VMEM((2, EMB, 128), jnp.float32),
            pltpu.VMEM((2, EMB, 128), jnp.float32),
            pltpu.VMEM((256,), jnp.float32),
            pltpu.VMEM((256,), jnp.float32),
            pltpu.VMEM((256,), jnp.float32),
            pltpu.VMEM((256,), jnp.float32),
            pltpu.VMEM((BPW * 16,), jnp.float32),
            pltpu.VMEM((BPW,), jnp.float32),
            pltpu.VMEM((16,), jnp.float32),
            pltpu.SemaphoreType.DMA,
            pltpu.SemaphoreType.DMA,
        ],
        compiler_params=pltpu.CompilerParams(
            needs_layout_passes=False, use_tc_tiling_on_sc=True),
    )(_mf_body)
    return f(u_id, i_id, uembT, ub1, iembT, ib1, mean)


def _tc_body(uw_ref, ul_ref, iw_ref, il_ref, *refs):
    (u0, u1, u2, u3, u4, u5, u6, u7,
     v0, v1, v2, v3, v4, v5, v6, v7,
     a0, a1, a2, a3, a4, a5, a6, a7,
     b0, b1, b2, b3, b4, b5, b6, b7,
     mean_ref, out_ref) = refs
    U = [u0, u1, u2, u3, u4, u5, u6, u7]
    V = [v0, v1, v2, v3, v4, v5, v6, v7]
    A = [a0, a1, a2, a3, a4, a5, a6, a7]
    Bb = [b0, b1, b2, b3, b4, b5, b6, b7]
    i = pl.program_id(0)
    lane = lax.broadcasted_iota(jnp.int32, (1, 128), 1)
    io8 = lax.broadcasted_iota(jnp.int32, (1, PPS), 1)
    acc = jnp.zeros((1, PPS), jnp.float32)
    for j in range(PPS):
        p = i * PPS + j
        cu = ul_ref[p]
        ci = il_ref[p]
        ohu = (lane == cu).astype(jnp.float32)
        ohi = (lane == ci).astype(jnp.float32)
        usel = jax.lax.dot_general(
            U[j][...], ohu.reshape(128, 1), (((1,), (0,)), ((), ())),
            preferred_element_type=jnp.float32)
        isel = jax.lax.dot_general(
            V[j][...], ohi.reshape(128, 1), (((1,), (0,)), ((), ())),
            preferred_element_type=jnp.float32)
        d = jnp.sum(usel * isel)
        row8 = lax.broadcasted_iota(jnp.int32, (8, 1), 0)
        ohru = (row8 == lax.rem(uw_ref[p], jnp.int32(8))).astype(jnp.float32)
        ohri = (row8 == lax.rem(iw_ref[p], jnp.int32(8))).astype(jnp.float32)
        ub = jnp.sum(A[j][...] * ohu * ohru)
        ib = jnp.sum(Bb[j][...] * ohi * ohri)
        acc = jnp.where(io8 == j, d + ub + ib, acc)
    out_ref[pl.ds(i, 1), :] = acc + mean_ref[0]


def _tc_call(uwin, ulane, iwin, ilane, uembT, ubp, iembT, ibp, mean):
    uspec = [pl.BlockSpec(
        (EMB, 128), functools.partial(
            lambda i, uw, ul, iw, il, jj: (0, uw[i * PPS + jj]), jj=j))
        for j in range(PPS)]
    ispec = [pl.BlockSpec(
        (EMB, 128), functools.partial(
            lambda i, uw, ul, iw, il, jj: (0, iw[i * PPS + jj]), jj=j))
        for j in range(PPS)]
    aspec = [pl.BlockSpec(
        (8, 128), functools.partial(
            lambda i, uw, ul, iw, il, jj: (uw[i * PPS + jj] // 8, 0), jj=j))
        for j in range(PPS)]
    bspec = [pl.BlockSpec(
        (8, 128), functools.partial(
            lambda i, uw, ul, iw, il, jj: (iw[i * PPS + jj] // 8, 0), jj=j))
        for j in range(PPS)]
    mspec = [pl.BlockSpec(memory_space=pltpu.SMEM)]
    grid_spec = pltpu.PrefetchScalarGridSpec(
        num_scalar_prefetch=4,
        grid=(TSTEPS,),
        in_specs=uspec + ispec + aspec + bspec + mspec,
        out_specs=pl.BlockSpec((TSTEPS, PPS), lambda i, *_: (0, 0)),
    )
    out2d = pl.pallas_call(
        _tc_body,
        grid_spec=grid_spec,
        out_shape=jax.ShapeDtypeStruct((TSTEPS, PPS), jnp.float32),
    )(uwin, ulane, iwin, ilane,
      *([uembT] * PPS), *([iembT] * PPS), *([ubp] * PPS), *([ibp] * PPS),
      mean)
    return out2d.reshape(TC_N)


@jax.jit
def kernel(u_id, i_id, user_emb_w, user_bias_w, item_emb_w, item_bias_w, mean):
    u_id = u_id.astype(jnp.int32)
    i_id = i_id.astype(jnp.int32)
    uembT = user_emb_w.T
    iembT = item_emb_w.T
    ub1 = user_bias_w.reshape(-1)
    ib1 = item_bias_w.reshape(-1)
    ubp = jnp.pad(ub1, (0, 7816 * 128 - 1000000)).reshape(7816, 128)
    ibp = jnp.pad(ib1, (0, 7816 * 128 - 1000000)).reshape(7816, 128)

    out_sc = _sc_call(u_id[:SC_N], i_id[:SC_N], uembT, ub1, iembT, ib1, mean)
    ut = u_id[SC_N:]
    it = i_id[SC_N:]
    out_tc = _tc_call(
        lax.shift_right_logical(ut, 7), lax.bitwise_and(ut, jnp.int32(127)),
        lax.shift_right_logical(it, 7), lax.bitwise_and(it, jnp.int32(127)),
        uembT, ubp, iembT, ibp, mean)
    return jnp.concatenate([out_sc, out_tc])


# speedup vs baseline: 2.9351x; 2.9351x over previous
import functools

import jax
import jax.numpy as jnp
from jax import lax
from jax.experimental import pallas as pl
from jax.experimental.pallas import tpu as pltpu
from jax.experimental.pallas import tpu_sc as plsc

B = 16384
EMB = 64
NC = 2
NS = 16
NW = NC * NS
BPW = B // NW
NBODY = BPW // 4
IDXPAD = BPW + 32


def _mf_body(u_id, i_id, uembT, ub1, iembT, ib1, mean, out,
             idx_u, idx_i, UW_A, IW_A, UW_B, IW_B,
             Ub_A, Ib_A, Ub_B, Ib_B, P, outv, meanv, semA, semB):
    wid = lax.axis_index("s") * NC + lax.axis_index("c")
    base = wid * BPW

    pltpu.sync_copy(u_id.at[pl.ds(base, BPW)], idx_u.at[pl.ds(0, BPW)])
    pltpu.sync_copy(i_id.at[pl.ds(base, BPW)], idx_i.at[pl.ds(0, BPW)])
    pltpu.sync_copy(mean, meanv.at[pl.ds(0, 1)])

    m16 = meanv[...][0] * 0.0625
    iot = lax.iota(jnp.int32, 16)

    def enq(uv, iv, UWx, IWx, Ubx, Ibx, semx):
        for j in range(2):
            uid = uv[j]
            iid = iv[j]
            cu = pl.multiple_of(lax.shift_right_logical(uid, 7) * 128, 128)
            ci = pl.multiple_of(lax.shift_right_logical(iid, 7) * 128, 128)
            pltpu.async_copy(uembT.at[:, pl.ds(cu, 128)], UWx.at[j], semx)
            pltpu.async_copy(iembT.at[:, pl.ds(ci, 128)], IWx.at[j], semx)
            pltpu.async_copy(ub1.at[pl.ds(cu, 128)],
                             Ubx.at[pl.ds(j * 128, 128)], semx)
            pltpu.async_copy(ib1.at[pl.ds(ci, 128)],
                             Ibx.at[pl.ds(j * 128, 128)], semx)

    def drain(UWx, IWx, Ubx, Ibx, semx):
        for j in range(2):
            pltpu.make_async_copy(
                uembT.at[:, pl.ds(0, 128)], UWx.at[j], semx).wait()
            pltpu.make_async_copy(
                iembT.at[:, pl.ds(0, 128)], IWx.at[j], semx).wait()
            pltpu.make_async_copy(
                ub1.at[pl.ds(0, 128)], Ubx.at[pl.ds(j * 128, 128)], semx).wait()
            pltpu.make_async_copy(
                ib1.at[pl.ds(0, 128)], Ibx.at[pl.ds(j * 128, 128)], semx).wait()

    def comp(p0, uv, iv, UWx, IWx, Ubx, Ibx):
        for j in range(2):
            uid = uv[j]
            iid = iv[j]
            luv = jnp.full((16,), 0, jnp.int32) + lax.bitwise_and(uid, jnp.int32(127))
            liv = jnp.full((16,), 0, jnp.int32) + lax.bitwise_and(iid, jnp.int32(127))
            jv = jnp.full((16,), j, jnp.int32)
            acc = jnp.zeros((16,), jnp.float32)
            for g in range(4):
                fv = g * 16 + iot
                uu = plsc.load_gather(UWx, [jv, fv, luv])
                vv = plsc.load_gather(IWx, [jv, fv, liv])
                acc = acc + uu * vv
            ubv = plsc.load_gather(Ubx, [jnp.full((16,), j * 128, jnp.int32) + luv])
            ibv = plsc.load_gather(Ibx, [jnp.full((16,), j * 128, jnp.int32) + liv])
            acc = acc + (ubv + ibv) * 0.0625 + m16
            P[pl.ds((p0 + j) * 16, 16)] = acc

    enq(idx_u[pl.ds(0, 16)], idx_i[pl.ds(0, 16)],
        UW_A, IW_A, Ub_A, Ib_A, semA)
    enq(idx_u[pl.ds(2, 16)], idx_i[pl.ds(2, 16)],
        UW_B, IW_B, Ub_B, Ib_B, semB)

    def body(k, carry):
        pA = k * 4
        uvA = idx_u[pl.ds(pA, 16)]
        ivA = idx_i[pl.ds(pA, 16)]
        drain(UW_A, IW_A, Ub_A, Ib_A, semA)
        comp(pA, uvA, ivA, UW_A, IW_A, Ub_A, Ib_A)

        @pl.when(k < NBODY - 1)
        def _():
            enq(idx_u[pl.ds(pA + 4, 16)], idx_i[pl.ds(pA + 4, 16)],
                UW_A, IW_A, Ub_A, Ib_A, semA)

        pB = k * 4 + 2
        uvB = idx_u[pl.ds(pB, 16)]
        ivB = idx_i[pl.ds(pB, 16)]
        drain(UW_B, IW_B, Ub_B, Ib_B, semB)
        comp(pB, uvB, ivB, UW_B, IW_B, Ub_B, Ib_B)

        @pl.when(k < NBODY - 1)
        def _():
            enq(idx_u[pl.ds(pB + 4, 16)], idx_i[pl.ds(pB + 4, 16)],
                UW_B, IW_B, Ub_B, Ib_B, semB)

        return carry

    lax.fori_loop(0, NBODY, body, 0)

    def phase2(b, carry):
        pvec = (b * 16 + iot) * 16
        acc = jnp.zeros((16,), jnp.float32)
        for l in range(16):
            acc = acc + plsc.load_gather(P, [pvec + jnp.full((16,), l, jnp.int32)])
        outv[pl.ds(b * 16, 16)] = acc
        return carry

    lax.fori_loop(0, BPW // 16, phase2, 0)

    pltpu.sync_copy(outv, out.at[pl.ds(base, BPW)])


@jax.jit
def kernel(u_id, i_id, user_emb_w, user_bias_w, item_emb_w, item_bias_w, mean):
    mesh = plsc.VectorSubcoreMesh(core_axis_name="c", subcore_axis_name="s")
    f = functools.partial(
        pl.kernel,
        out_type=jax.ShapeDtypeStruct((B,), jnp.float32),
        mesh=mesh,
        scratch_types=[
            pltpu.VMEM((IDXPAD,), jnp.int32),
            pltpu.VMEM((IDXPAD,), jnp.int32),
            pltpu.VMEM((2, EMB, 128), jnp.float32),
            pltpu.VMEM((2, EMB, 128), jnp.float32),
            pltpu.VMEM((2, EMB, 128), jnp.float32),
            pltpu.VMEM((2, EMB, 128), jnp.float32),
            pltpu.VMEM((256,), jnp.float32),
            pltpu.VMEM((256,), jnp.float32),
            pltpu.VMEM((256,), jnp.float32),
            pltpu.VMEM((256,), jnp.float32),
            pltpu.VMEM((BPW * 16,), jnp.float32),
            pltpu.VMEM((BPW,), jnp.float32),
            pltpu.VMEM((16,), jnp.float32),
            pltpu.SemaphoreType.DMA,
            pltpu.SemaphoreType.DMA,
        ],
        compiler_params=pltpu.CompilerParams(
            needs_layout_passes=False, use_tc_tiling_on_sc=True),
    )(_mf_body)
    return f(u_id.astype(jnp.int32), i_id.astype(jnp.int32),
             user_emb_w.T, user_bias_w.reshape(-1),
             item_emb_w.T, item_bias_w.reshape(-1), mean)
